# baseline (speedup 1.0000x reference)
import numpy as np
import jax
import jax.numpy as jnp
from jax import lax
from jax.experimental import pallas as pl
from jax.experimental.pallas import tpu as pltpu
from jax.experimental.pallas import tpu_sc as plsc

_N_EMBED = 256
_CH = 86
_NN = (4000, 30000, 150000)
_SCALES = (16, 32, 64)
_DOFF = (0, 16, 48)
_ROW0 = (0, 4000, 34000)
_NC = 1
_NW = 16 * _NC
_L = 16


def _round_up(x, m):
    return (x + m - 1) // m * m


_BROWS = 128
_CW = tuple(_round_up(-(-n // _NW), 2 * _BROWS) for n in _NN)
_NB2 = tuple(c // (2 * _BROWS) for c in _CW)


def _axis_table():
    blocks = []
    for scale in _SCALES:
        inv_freq = 1.0 / (
            10000.0 ** (np.arange(0, _CH, 2, dtype=np.float64) / _CH)
        )
        pos = np.arange(scale, dtype=np.float64)
        sin_inp = pos[:, None] * inv_freq[None, :]
        emb = np.stack([np.sin(sin_inp), np.cos(sin_inp)], axis=-1)
        ax = emb.reshape(scale, _CH).astype(np.float32)
        blocks.append(np.concatenate([ax, ax, ax[:, : _N_EMBED - 2 * _CH]], axis=1))
    return np.concatenate(blocks, axis=0)


_TAB_NP = _axis_table()


def _sc_body(xyz4, xyz5, xyz6, tab_hbm, out_hbm, tab_v, cbuf, obufs, sems):
    cid = lax.axis_index("c")
    sid = lax.axis_index("s")
    wid = sid * _NC + cid

    pltpu.sync_copy(tab_hbm, tab_v)
    lane = lax.iota(jnp.int32, _L)
    olane = lane * _N_EMBED

    xyz_refs = (xyz4, xyz5, xyz6)
    for di in range(3):
        n_d, c_d = _NN[di], _CW[di]
        start = jnp.minimum(wid * c_d, n_d - c_d)
        pltpu.sync_copy(
            xyz_refs[di].at[pl.ds(start * 3, c_d * 3)],
            cbuf.at[pl.ds(0, c_d * 3)],
        )
        row0 = _ROW0[di] + start

        def outer(it, carry, di=di, row0=row0):
            for b in range(2):
                obuf = obufs[b]
                sem = sems[b]
                blk = it * 2 + b

                @pl.when(it > 0)
                def _wait():
                    pltpu.make_async_copy(
                        obuf, out_hbm.at[pl.ds(0, _BROWS * _N_EMBED)], sem
                    ).wait()

                def quad(s, di=di, blk=blk, obuf=obuf):
                    cv = cbuf[pl.ds(blk * (3 * _BROWS) + s * 3, _L)]

                    for n in range(1):
                        xb = (cv[3 * n] + _DOFF[di]) * _N_EMBED
                        yb = (cv[3 * n + 1] + _DOFF[di]) * _N_EMBED
                        zb = (cv[3 * n + 2] + _DOFF[di]) * _N_EMBED
                        ob = s * _N_EMBED
                        for j in range(16):
                            off = _L * j
                            if j < 5:
                                src = tab_v[pl.ds(xb + off, _L)]
                            elif j == 5:
                                vx = tab_v[pl.ds(xb + off, _L)]
                                vy = tab_v[pl.ds(yb + off, _L)]
                                src = jnp.where(lane < _CH - off, vx, vy)
                            elif j < 10:
                                src = tab_v[pl.ds(yb + off, _L)]
                            elif j == 10:
                                vy = tab_v[pl.ds(yb + off, _L)]
                                vz = tab_v[pl.ds(zb + off, _L)]
                                src = jnp.where(lane < 2 * _CH - off, vy, vz)
                            else:
                                src = tab_v[pl.ds(zb + off, _L)]
                            obuf[pl.ds(ob + off, _L)] = src

                plsc.parallel_loop(0, _BROWS, unroll=1)(quad)

                pltpu.make_async_copy(
                    obuf,
                    out_hbm.at[
                        pl.ds((row0 + blk * _BROWS) * _N_EMBED,
                              _BROWS * _N_EMBED)
                    ],
                    sem,
                ).start()
            return carry

        lax.fori_loop(0, _NB2[di], outer, None)

        for b in range(2):
            pltpu.make_async_copy(
                obufs[b], out_hbm.at[pl.ds(0, _BROWS * _N_EMBED)], sems[b]
            ).wait()


def _sc_kernel_body(xyz4, xyz5, xyz6, tab_hbm, out_hbm, tab_v, cbuf,
                    ob0, ob1, sem0, sem1):
    _sc_body(xyz4, xyz5, xyz6, tab_hbm, out_hbm, tab_v, cbuf,
             (ob0, ob1), (sem0, sem1))


def _freq_phase():
    j = np.arange(_N_EMBED) % _CH
    inv_freq = 1.0 / (10000.0 ** ((2 * (j // 2)) / _CH))
    phase = np.where(j % 2 == 1, np.pi / 2, 0.0)
    return (
        inv_freq.astype(np.float32).reshape(1, -1),
        phase.astype(np.float32).reshape(1, -1),
    )


_FREQ_NP, _PHASE_NP = _freq_phase()
_TC_BLK = 1000


def _tc_body(b1, b2, x_ref, y_ref, z_ref, bias_ref, freq_ref, phase_ref,
             out_ref):
    cols = lax.broadcasted_iota(jnp.int32, (_TC_BLK, _N_EMBED), 1)
    x = x_ref[...].astype(jnp.float32)
    y = y_ref[...].astype(jnp.float32)
    z = z_ref[...].astype(jnp.float32)
    coord = jnp.where(cols < _CH, x, jnp.where(cols < 2 * _CH, y, z))
    i = pl.program_id(0)
    bias = jnp.where(
        i < b1, bias_ref[0:1, :],
        jnp.where(i < b2, bias_ref[1:2, :], bias_ref[2:3, :]),
    )
    ang = coord * freq_ref[...] + phase_ref[...]
    out_ref[...] = jnp.sin(ang) + bias


def _tc_rows(x_all, y_all, z_all, bias, n_rows):
    total = sum(n_rows)
    nblk = [n // _TC_BLK for n in n_rows]
    b1, b2 = nblk[0], nblk[0] + nblk[1]

    cspec = pl.BlockSpec((_TC_BLK, 1), lambda i: (i, 0))
    import functools as _ft
    return pl.pallas_call(
        _ft.partial(_tc_body, b1, b2),
        grid=(total // _TC_BLK,),
        in_specs=[
            cspec,
            cspec,
            cspec,
            pl.BlockSpec((3, _N_EMBED), lambda i: (0, 0)),
            pl.BlockSpec((1, _N_EMBED), lambda i: (0, 0)),
            pl.BlockSpec((1, _N_EMBED), lambda i: (0, 0)),
        ],
        out_specs=pl.BlockSpec((_TC_BLK, _N_EMBED), lambda i: (i, 0)),
        out_shape=jax.ShapeDtypeStruct((total, _N_EMBED), jnp.float32),
    )(x_all[:, None], y_all[:, None], z_all[:, None], bias,
      jnp.asarray(_FREQ_NP), jnp.asarray(_PHASE_NP))


def kernel(xyz_d4, xyz_d5, xyz_d6, depth_emb_weight, depth_low, depth_high):
    total = sum(_NN)
    tab = jnp.asarray(_TAB_NP) + jnp.repeat(
        depth_emb_weight.astype(jnp.float32),
        jnp.array(_SCALES),
        axis=0,
        total_repeat_length=112,
    )
    tab_flat = tab.reshape(-1)

    flats = [x.astype(jnp.int32).reshape(-1) for x in (xyz_d4, xyz_d5, xyz_d6)]

    mesh = plsc.VectorSubcoreMesh(
        core_axis_name="c", subcore_axis_name="s", num_cores=_NC)
    run = pl.kernel(
        _sc_kernel_body,
        out_type=jax.ShapeDtypeStruct((total * _N_EMBED,), jnp.float32),
        mesh=mesh,
        compiler_params=pltpu.CompilerParams(needs_layout_passes=False),
        scratch_types=[
            pltpu.VMEM((112 * _N_EMBED,), jnp.float32),
            pltpu.VMEM((3 * _CW[2] + _L,), jnp.int32),
            pltpu.VMEM((_BROWS * _N_EMBED,), jnp.float32),
            pltpu.VMEM((_BROWS * _N_EMBED,), jnp.float32),
            pltpu.SemaphoreType.DMA,
            pltpu.SemaphoreType.DMA,
        ],
    )
    out = run(flats[0], flats[1], flats[2], tab_flat)
    return out.reshape(total, _N_EMBED)


def _kernel_tc_only(xyz_d4, xyz_d5, xyz_d6, depth_emb_weight, dl, dh):
    xs = [x.astype(jnp.int32) for x in (xyz_d4, xyz_d5, xyz_d6)]
    x_all = jnp.concatenate([x[:, 0] for x in xs])
    y_all = jnp.concatenate([x[:, 1] for x in xs])
    z_all = jnp.concatenate([x[:, 2] for x in xs])
    return _tc_rows(x_all, y_all, z_all,
                    depth_emb_weight.astype(jnp.float32), _NN)


# speedup vs baseline: 1.1403x; 1.1403x over previous
import numpy as np
import jax
import jax.numpy as jnp
from jax import lax
from jax.experimental import pallas as pl
from jax.experimental.pallas import tpu as pltpu
from jax.experimental.pallas import tpu_sc as plsc

_N_EMBED = 256
_CH = 86
_NN = (4000, 30000, 150000)
_SCALES = (16, 32, 64)
_DOFF = (0, 16, 48)
_ROW0 = (0, 4000, 34000)
_NC = 2
_NW = 16 * _NC
_L = 16


def _round_up(x, m):
    return (x + m - 1) // m * m


_BROWS = 128
_CW = tuple(_round_up(-(-n // _NW), 2 * _BROWS) for n in _NN)
_NB2 = tuple(c // (2 * _BROWS) for c in _CW)


def _axis_table():
    blocks = []
    for scale in _SCALES:
        inv_freq = 1.0 / (
            10000.0 ** (np.arange(0, _CH, 2, dtype=np.float64) / _CH)
        )
        pos = np.arange(scale, dtype=np.float64)
        sin_inp = pos[:, None] * inv_freq[None, :]
        emb = np.stack([np.sin(sin_inp), np.cos(sin_inp)], axis=-1)
        ax = emb.reshape(scale, _CH).astype(np.float32)
        blocks.append(np.concatenate([ax, ax, ax[:, : _N_EMBED - 2 * _CH]], axis=1))
    return np.concatenate(blocks, axis=0)


_TAB_NP = _axis_table()


def _sc_body(xyz4, xyz5, xyz6, tab_hbm, out_hbm, tab_v, cbuf, obufs, sems):
    cid = lax.axis_index("c")
    sid = lax.axis_index("s")
    wid = sid * _NC + cid

    pltpu.sync_copy(tab_hbm, tab_v)
    lane = lax.iota(jnp.int32, _L)

    xyz_refs = (xyz4, xyz5, xyz6)
    for di in range(3):
        n_d, c_d = _NN[di], _CW[di]
        start = jnp.minimum(wid * c_d, n_d - c_d)
        pltpu.sync_copy(
            xyz_refs[di].at[pl.ds(start * 3, c_d * 3)],
            cbuf.at[pl.ds(0, c_d * 3)],
        )
        row0 = _ROW0[di] + start

        def outer(it, carry, di=di, row0=row0):
            for b in range(2):
                obuf = obufs[b]
                sem = sems[b]
                blk = it * 2 + b

                @pl.when(it > 0)
                def _wait():
                    pltpu.make_async_copy(
                        obuf, out_hbm.at[pl.ds(0, _BROWS * _N_EMBED)], sem
                    ).wait()

                def node_iter(s, di=di, blk=blk, obuf=obuf):
                    cv = cbuf[pl.ds(blk * (3 * _BROWS) + s * 3, _L)]

                    for n in range(1):
                        xb = (cv[3 * n] + _DOFF[di]) * _N_EMBED
                        yb = (cv[3 * n + 1] + _DOFF[di]) * _N_EMBED
                        zb = (cv[3 * n + 2] + _DOFF[di]) * _N_EMBED
                        ob = s * _N_EMBED
                        for j in range(16):
                            off = _L * j
                            if j < 5:
                                src = tab_v[pl.ds(xb + off, _L)]
                            elif j == 5:
                                vx = tab_v[pl.ds(xb + off, _L)]
                                vy = tab_v[pl.ds(yb + off, _L)]
                                src = jnp.where(lane < _CH - off, vx, vy)
                            elif j < 10:
                                src = tab_v[pl.ds(yb + off, _L)]
                            elif j == 10:
                                vy = tab_v[pl.ds(yb + off, _L)]
                                vz = tab_v[pl.ds(zb + off, _L)]
                                src = jnp.where(lane < 2 * _CH - off, vy, vz)
                            else:
                                src = tab_v[pl.ds(zb + off, _L)]
                            obuf[pl.ds(ob + off, _L)] = src

                plsc.parallel_loop(0, _BROWS, unroll=1)(node_iter)

                pltpu.make_async_copy(
                    obuf,
                    out_hbm.at[
                        pl.ds((row0 + blk * _BROWS) * _N_EMBED,
                              _BROWS * _N_EMBED)
                    ],
                    sem,
                ).start()
            return carry

        lax.fori_loop(0, _NB2[di], outer, None)

        for b in range(2):
            pltpu.make_async_copy(
                obufs[b], out_hbm.at[pl.ds(0, _BROWS * _N_EMBED)], sems[b]
            ).wait()


def _sc_kernel_body(xyz4, xyz5, xyz6, tab_hbm, out_hbm, tab_v, cbuf,
                    ob0, ob1, sem0, sem1):
    _sc_body(xyz4, xyz5, xyz6, tab_hbm, out_hbm, tab_v, cbuf,
             (ob0, ob1), (sem0, sem1))


def kernel(xyz_d4, xyz_d5, xyz_d6, depth_emb_weight, depth_low, depth_high):
    total = sum(_NN)
    tab = jnp.asarray(_TAB_NP) + jnp.repeat(
        depth_emb_weight.astype(jnp.float32),
        jnp.array(_SCALES),
        axis=0,
        total_repeat_length=112,
    )
    tab_flat = tab.reshape(-1)

    flats = [x.astype(jnp.int32).reshape(-1) for x in (xyz_d4, xyz_d5, xyz_d6)]

    mesh = plsc.VectorSubcoreMesh(
        core_axis_name="c", subcore_axis_name="s", num_cores=_NC)
    run = pl.kernel(
        _sc_kernel_body,
        out_type=jax.ShapeDtypeStruct((total * _N_EMBED,), jnp.float32),
        mesh=mesh,
        compiler_params=pltpu.CompilerParams(needs_layout_passes=False),
        scratch_types=[
            pltpu.VMEM((112 * _N_EMBED,), jnp.float32),
            pltpu.VMEM((3 * _CW[2] + _L,), jnp.int32),
            pltpu.VMEM((_BROWS * _N_EMBED,), jnp.float32),
            pltpu.VMEM((_BROWS * _N_EMBED,), jnp.float32),
            pltpu.SemaphoreType.DMA,
            pltpu.SemaphoreType.DMA,
        ],
    )
    out = run(flats[0], flats[1], flats[2], tab_flat)
    return out.reshape(total, _N_EMBED)
